# baseline (speedup 1.0000x reference)
import functools

import jax
import jax.numpy as jnp
from jax import lax
from jax.experimental import pallas as pl
from jax.experimental.pallas import tpu as pltpu
from jax.experimental.pallas import tpu_sc as plsc

N = 100000
C = 128

NC = 2
NS = 16
NW = NC * NS
NP = 102400
ROWS_PER_W = NP // 8
SC_CHUNK = 128

BLK = 1000


NBUF = 4
R0 = 18432
R1 = 7168


def _sc_gather_body(cols_hbm, x_hbm, g_hbm, idx_all, rows_v, gsem, ssem):
    cid = lax.axis_index("c")
    sid = lax.axis_index("s")
    slot = sid // 4
    j = sid % 4

    def pipeline(rows, base):
        nch = rows // SC_CHUNK
        nq = nch // NBUF
        idx_w = idx_all.at[pl.ds(0, rows)]
        pltpu.sync_copy(cols_hbm.at[slot, pl.ds(base, rows)], idx_w)

        def idx_at(k):
            return idx_all.at[pl.ds(pl.multiple_of(k * SC_CHUNK, SC_CHUNK),
                                    SC_CHUNK)]

        def gather(k, b):
            pltpu.async_copy(x_hbm.at[idx_at(k)], rows_v.at[b], gsem.at[b])

        def gather_wait(b):
            pltpu.make_async_copy(
                x_hbm.at[idx_at(0)], rows_v.at[b], gsem.at[b]).wait()

        def scatter_descr(k, b):
            off = pl.multiple_of(base + k * SC_CHUNK, SC_CHUNK)
            return pltpu.make_async_copy(
                rows_v.at[b], g_hbm.at[slot, pl.ds(off, SC_CHUNK)],
                ssem.at[b])

        for b in range(NBUF):
            gather(b, b)

        def body(q, carry):
            for b in range(NBUF):
                k = q * NBUF + b
                gather_wait(b)
                scatter_descr(k, b).start()

                @pl.when(q < nq - 1)
                def _():
                    scatter_descr(k, b).wait()
                    gather(k + NBUF, b)

            return carry

        lax.fori_loop(0, nq, body, 0)
        for b in range(NBUF):
            scatter_descr(nch - NBUF + b, b).wait()

    @pl.when(cid == 1)
    def _():
        pipeline(R0, j * R0)

    @pl.when(cid == 0)
    def _():
        pipeline(R1, 4 * R0 + j * R1)


@functools.cache
def _sc_gather():
    return pl.kernel(
        _sc_gather_body,
        out_type=jax.ShapeDtypeStruct((4, NP, C), jnp.float32),
        mesh=plsc.VectorSubcoreMesh(
            core_axis_name="c", subcore_axis_name="s",
            num_cores=NC, num_subcores=NS,
        ),
        scratch_types=[
            pltpu.VMEM((R0,), jnp.int32),
            pltpu.VMEM((NBUF, SC_CHUNK, C), jnp.float32),
            pltpu.SemaphoreType.DMA((NBUF,)),
            pltpu.SemaphoreType.DMA((NBUF,)),
        ],
    )


def _tc_fused_body(x_ref, g_ref, v1_ref, v2_ref, v3_ref, v4_ref, W_ref, b_ref,
                   ls_ref, lb_ref, o_ref):
    x = x_ref[...]
    a = g_ref[0] * v1_ref[...]
    bb = g_ref[1] * v2_ref[...]
    c = g_ref[2] * v3_ref[...]
    d = g_ref[3] * v4_ref[...]
    patch = jnp.concatenate(
        [x, jnp.abs(a - c), a + c, jnp.abs(bb - d), bb + d], axis=-1)
    y = jnp.dot(patch, W_ref[...], preferred_element_type=jnp.float32)
    y = y + b_ref[...]
    mu = jnp.mean(y, axis=-1, keepdims=True)
    yc = y - mu
    var = jnp.mean(yc * yc, axis=-1, keepdims=True)
    y = yc * lax.rsqrt(var + 1e-5) * ls_ref[...] + lb_ref[...]
    y = y + x
    o_ref[...] = 0.5 * y * (1.0 + lax.erf(y * 0.7071067811865476))


def kernel(x, col1, col2, col3, col4, val1, val2, val3, val4, W, b, ln_scale,
           ln_bias):
    cols = jnp.stack([col1, col2, col3, col4]).astype(jnp.int32)
    cols = jnp.pad(cols, ((0, 0), (0, NP - N)))
    g = _sc_gather()(cols, x)

    grid = (N // BLK,)
    out = pl.pallas_call(
        _tc_fused_body,
        grid=grid,
        in_specs=[
            pl.BlockSpec((BLK, C), lambda i: (i, 0)),
            pl.BlockSpec((4, BLK, C), lambda i: (0, i, 0)),
            pl.BlockSpec((BLK, 1), lambda i: (i, 0)),
            pl.BlockSpec((BLK, 1), lambda i: (i, 0)),
            pl.BlockSpec((BLK, 1), lambda i: (i, 0)),
            pl.BlockSpec((BLK, 1), lambda i: (i, 0)),
            pl.BlockSpec((5 * C, C), lambda i: (0, 0)),
            pl.BlockSpec((1, C), lambda i: (0, 0)),
            pl.BlockSpec((1, C), lambda i: (0, 0)),
            pl.BlockSpec((1, C), lambda i: (0, 0)),
        ],
        out_specs=pl.BlockSpec((BLK, C), lambda i: (i, 0)),
        out_shape=jax.ShapeDtypeStruct((N, C), jnp.float32),
    )(x, g, val1[:, None], val2[:, None], val3[:, None], val4[:, None],
      W, b[None, :], ln_scale[None, :], ln_bias[None, :])
    return out


# speedup vs baseline: 1.0004x; 1.0004x over previous
import functools

import jax
import jax.numpy as jnp
from jax import lax
from jax.experimental import pallas as pl
from jax.experimental.pallas import tpu as pltpu
from jax.experimental.pallas import tpu_sc as plsc

N = 100000
C = 128

NC = 2
NS = 16
NW = NC * NS
NP = 102400
ROWS_PER_W = NP // 8
SC_CHUNK = 128

BLK = 1000


NBUF = 5
R0 = 18560
R1 = 7040


def _sc_gather_body(cols_hbm, x_hbm, g_hbm, idx_all, rows_v, gsem, ssem):
    cid = lax.axis_index("c")
    sid = lax.axis_index("s")
    slot = sid // 4
    j = sid % 4

    fast = cid == 1
    nch = jnp.where(fast, R0 // SC_CHUNK, R1 // SC_CHUNK)
    nq = nch // NBUF
    base = jnp.where(fast, j * R0, 4 * R0 + j * R1)

    pltpu.sync_copy(cols_hbm.at[slot, pl.ds(base, R0)], idx_all)

    def idx_at(k):
        return idx_all.at[pl.ds(pl.multiple_of(k * SC_CHUNK, SC_CHUNK),
                                SC_CHUNK)]

    def gather(k, b):
        pltpu.async_copy(x_hbm.at[idx_at(k)], rows_v.at[b], gsem.at[b])

    def gather_wait(b):
        pltpu.make_async_copy(
            x_hbm.at[idx_at(0)], rows_v.at[b], gsem.at[b]).wait()

    def scatter_descr(k, b):
        off = pl.multiple_of(base + k * SC_CHUNK, SC_CHUNK)
        return pltpu.make_async_copy(
            rows_v.at[b], g_hbm.at[slot, pl.ds(off, SC_CHUNK)], ssem.at[b])

    for b in range(NBUF):
        gather(b, b)

    def body(q, carry):
        for b in range(NBUF):
            k = q * NBUF + b
            gather_wait(b)
            scatter_descr(k, b).start()

            @pl.when(q < nq - 1)
            def _():
                scatter_descr(k, b).wait()
                gather(k + NBUF, b)

        return carry

    lax.fori_loop(0, nq, body, 0)
    for b in range(NBUF):
        scatter_descr(nch - NBUF + b, b).wait()


@functools.cache
def _sc_gather():
    return pl.kernel(
        _sc_gather_body,
        out_type=jax.ShapeDtypeStruct((4, NP, C), jnp.float32),
        mesh=plsc.VectorSubcoreMesh(
            core_axis_name="c", subcore_axis_name="s",
            num_cores=NC, num_subcores=NS,
        ),
        scratch_types=[
            pltpu.VMEM((R0,), jnp.int32),
            pltpu.VMEM((NBUF, SC_CHUNK, C), jnp.float32),
            pltpu.SemaphoreType.DMA((NBUF,)),
            pltpu.SemaphoreType.DMA((NBUF,)),
        ],
    )


def _tc_fused_body(x_ref, g_ref, v1_ref, v2_ref, v3_ref, v4_ref, W_ref, b_ref,
                   ls_ref, lb_ref, o_ref):
    x = x_ref[...]
    a = g_ref[0] * v1_ref[...]
    bb = g_ref[1] * v2_ref[...]
    c = g_ref[2] * v3_ref[...]
    d = g_ref[3] * v4_ref[...]
    patch = jnp.concatenate(
        [x, jnp.abs(a - c), a + c, jnp.abs(bb - d), bb + d], axis=-1)
    y = jnp.dot(patch, W_ref[...], preferred_element_type=jnp.float32)
    y = y + b_ref[...]
    mu = jnp.mean(y, axis=-1, keepdims=True)
    yc = y - mu
    var = jnp.mean(yc * yc, axis=-1, keepdims=True)
    y = yc * lax.rsqrt(var + 1e-5) * ls_ref[...] + lb_ref[...]
    y = y + x
    o_ref[...] = 0.5 * y * (1.0 + lax.erf(y * 0.7071067811865476))


def kernel(x, col1, col2, col3, col4, val1, val2, val3, val4, W, b, ln_scale,
           ln_bias):
    cols = jnp.stack([col1, col2, col3, col4]).astype(jnp.int32)
    cols = jnp.pad(cols, ((0, 0), (0, NP - N + R0 - R1)))
    g = _sc_gather()(cols, x)

    grid = (N // BLK,)
    out = pl.pallas_call(
        _tc_fused_body,
        grid=grid,
        in_specs=[
            pl.BlockSpec((BLK, C), lambda i: (i, 0)),
            pl.BlockSpec((4, BLK, C), lambda i: (0, i, 0)),
            pl.BlockSpec((BLK, 1), lambda i: (i, 0)),
            pl.BlockSpec((BLK, 1), lambda i: (i, 0)),
            pl.BlockSpec((BLK, 1), lambda i: (i, 0)),
            pl.BlockSpec((BLK, 1), lambda i: (i, 0)),
            pl.BlockSpec((5 * C, C), lambda i: (0, 0)),
            pl.BlockSpec((1, C), lambda i: (0, 0)),
            pl.BlockSpec((1, C), lambda i: (0, 0)),
            pl.BlockSpec((1, C), lambda i: (0, 0)),
        ],
        out_specs=pl.BlockSpec((BLK, C), lambda i: (i, 0)),
        out_shape=jax.ShapeDtypeStruct((N, C), jnp.float32),
    )(x, g, val1[:, None], val2[:, None], val3[:, None], val4[:, None],
      W, b[None, :], ln_scale[None, :], ln_bias[None, :])
    return out


# speedup vs baseline: 1.0172x; 1.0168x over previous
import functools

import jax
import jax.numpy as jnp
from jax import lax
from jax.experimental import pallas as pl
from jax.experimental.pallas import tpu as pltpu
from jax.experimental.pallas import tpu_sc as plsc

N = 100000
C = 128

NC = 2
NS = 16
NP = 102400
NHALF = 2
NP2 = NP // NHALF
ROWS_PER_W = NP2 // 8
SC_CHUNK = 128
NCH = ROWS_PER_W // SC_CHUNK
NBUF = 5
NQ = NCH // NBUF

BLK = 800


def _sc_gather_body(h, cols_hbm, x_hbm, g_hbm, idx_all, rows_v, gsem, ssem):
    cid = lax.axis_index("c")
    sid = lax.axis_index("s")
    wid = sid * NC + cid
    slot = wid // 8
    sub = wid % 8
    base = h * NP2 + sub * ROWS_PER_W
    obase = sub * ROWS_PER_W

    pltpu.sync_copy(cols_hbm.at[slot, pl.ds(base, ROWS_PER_W)], idx_all)

    def idx_at(k):
        return idx_all.at[pl.ds(pl.multiple_of(k * SC_CHUNK, SC_CHUNK),
                                SC_CHUNK)]

    def gather(k, b):
        pltpu.async_copy(x_hbm.at[idx_at(k)], rows_v.at[b], gsem.at[b])

    def gather_wait(b):
        pltpu.make_async_copy(
            x_hbm.at[idx_at(0)], rows_v.at[b], gsem.at[b]).wait()

    def scatter_descr(k, b):
        off = pl.multiple_of(obase + k * SC_CHUNK, SC_CHUNK)
        return pltpu.make_async_copy(
            rows_v.at[b], g_hbm.at[slot, pl.ds(off, SC_CHUNK)], ssem.at[b])

    for b in range(NBUF):
        gather(b, b)

    def body(q, carry):
        for b in range(NBUF):
            k = q * NBUF + b
            gather_wait(b)
            scatter_descr(k, b).start()

            @pl.when(q < NQ - 1)
            def _():
                scatter_descr(k, b).wait()
                gather(k + NBUF, b)

        return carry

    lax.fori_loop(0, NQ, body, 0)
    for b in range(NBUF):
        scatter_descr(NCH - NBUF + b, b).wait()


@functools.cache
def _sc_gather(h):
    return pl.kernel(
        functools.partial(_sc_gather_body, h),
        out_type=jax.ShapeDtypeStruct((4, NP2, C), jnp.float32),
        mesh=plsc.VectorSubcoreMesh(
            core_axis_name="c", subcore_axis_name="s",
            num_cores=NC, num_subcores=NS,
        ),
        scratch_types=[
            pltpu.VMEM((ROWS_PER_W,), jnp.int32),
            pltpu.VMEM((NBUF, SC_CHUNK, C), jnp.float32),
            pltpu.SemaphoreType.DMA((NBUF,)),
            pltpu.SemaphoreType.DMA((NBUF,)),
        ],
    )


def _tc_fused_body(x_ref, g_ref, v1_ref, v2_ref, v3_ref, v4_ref, W_ref, b_ref,
                   ls_ref, lb_ref, o_ref):
    x = x_ref[...]
    a = g_ref[0] * v1_ref[...]
    bb = g_ref[1] * v2_ref[...]
    c = g_ref[2] * v3_ref[...]
    d = g_ref[3] * v4_ref[...]
    patch = jnp.concatenate(
        [x, jnp.abs(a - c), a + c, jnp.abs(bb - d), bb + d], axis=-1)
    y = jnp.dot(patch, W_ref[...], preferred_element_type=jnp.float32)
    y = y + b_ref[...]
    mu = jnp.mean(y, axis=-1, keepdims=True)
    yc = y - mu
    var = jnp.mean(yc * yc, axis=-1, keepdims=True)
    y = yc * lax.rsqrt(var + 1e-5) * ls_ref[...] + lb_ref[...]
    y = y + x
    o_ref[...] = 0.5 * y * (1.0 + lax.erf(y * 0.7071067811865476))


def _tc_call(h, nblk, x, g, v1, v2, v3, v4, W, b2, ls2, lb2):
    off = h * (NP2 // BLK)

    def shifted(i):
        return (i + off, 0)

    return pl.pallas_call(
        _tc_fused_body,
        grid=(nblk,),
        in_specs=[
            pl.BlockSpec((BLK, C), shifted),
            pl.BlockSpec((4, BLK, C), lambda i: (0, i, 0)),
            pl.BlockSpec((BLK, 1), shifted),
            pl.BlockSpec((BLK, 1), shifted),
            pl.BlockSpec((BLK, 1), shifted),
            pl.BlockSpec((BLK, 1), shifted),
            pl.BlockSpec((5 * C, C), lambda i: (0, 0)),
            pl.BlockSpec((1, C), lambda i: (0, 0)),
            pl.BlockSpec((1, C), lambda i: (0, 0)),
            pl.BlockSpec((1, C), lambda i: (0, 0)),
        ],
        out_specs=pl.BlockSpec((BLK, C), lambda i: (i, 0)),
        out_shape=jax.ShapeDtypeStruct((nblk * BLK, C), jnp.float32),
    )(x, g, v1, v2, v3, v4, W, b2, ls2, lb2)


def kernel(x, col1, col2, col3, col4, val1, val2, val3, val4, W, b, ln_scale,
           ln_bias):
    cols = jnp.stack([col1, col2, col3, col4]).astype(jnp.int32)
    cols = jnp.pad(cols, ((0, 0), (0, NP - N)))
    g0 = _sc_gather(0)(cols, x)
    g1 = _sc_gather(1)(cols, x)

    v1, v2, v3, v4 = (val1[:, None], val2[:, None], val3[:, None],
                      val4[:, None])
    b2, ls2, lb2 = b[None, :], ln_scale[None, :], ln_bias[None, :]
    out0 = _tc_call(0, NP2 // BLK, x, g0, v1, v2, v3, v4, W, b2, ls2, lb2)
    out1 = _tc_call(1, (N - NP2) // BLK, x, g1, v1, v2, v3, v4, W, b2, ls2,
                    lb2)
    return jnp.concatenate([out0, out1], axis=0)


# speedup vs baseline: 1.2221x; 1.2014x over previous
import functools

import jax
import jax.numpy as jnp
from jax import lax
from jax.experimental import pallas as pl
from jax.experimental.pallas import tpu as pltpu
from jax.experimental.pallas import tpu_sc as plsc

N = 100000
C = 128

NC = 2
NS = 16
NP = 102400
ROWS_PER_W = NP // 8
SC_CHUNK = 128
NCH = ROWS_PER_W // SC_CHUNK
NBUF = 5
NQ = NCH // NBUF

BLK = 1024


def _sc_gather_body(cols_hbm, x_hbm, g_hbm, idx_all, rows_v, gsem, ssem):
    cid = lax.axis_index("c")
    sid = lax.axis_index("s")
    wid = sid * NC + cid
    slot = wid // 8
    sub = wid % 8
    base = sub * ROWS_PER_W

    pltpu.sync_copy(cols_hbm.at[slot, pl.ds(base, ROWS_PER_W)], idx_all)

    def idx_at(k):
        return idx_all.at[pl.ds(pl.multiple_of(k * SC_CHUNK, SC_CHUNK),
                                SC_CHUNK)]

    def gather(k, b):
        pltpu.async_copy(x_hbm.at[idx_at(k)], rows_v.at[b], gsem.at[b])

    def gather_wait(b):
        pltpu.make_async_copy(
            x_hbm.at[idx_at(0)], rows_v.at[b], gsem.at[b]).wait()

    def scatter_descr(k, b):
        off = pl.multiple_of(base + k * SC_CHUNK, SC_CHUNK)
        return pltpu.make_async_copy(
            rows_v.at[b], g_hbm.at[slot, pl.ds(off, SC_CHUNK)], ssem.at[b])

    for b in range(NBUF):
        gather(b, b)

    def body(q, carry):
        for b in range(NBUF):
            k = q * NBUF + b
            gather_wait(b)
            scatter_descr(k, b).start()

            @pl.when(q < NQ - 1)
            def _():
                scatter_descr(k, b).wait()
                gather(k + NBUF, b)

        return carry

    lax.fori_loop(0, NQ, body, 0)
    for b in range(NBUF):
        scatter_descr(NCH - NBUF + b, b).wait()


@functools.cache
def _sc_gather():
    return pl.kernel(
        _sc_gather_body,
        out_type=jax.ShapeDtypeStruct((4, NP, C), jnp.float32),
        mesh=plsc.VectorSubcoreMesh(
            core_axis_name="c", subcore_axis_name="s",
            num_cores=NC, num_subcores=NS,
        ),
        scratch_types=[
            pltpu.VMEM((ROWS_PER_W,), jnp.int32),
            pltpu.VMEM((NBUF, SC_CHUNK, C), jnp.float32),
            pltpu.SemaphoreType.DMA((NBUF,)),
            pltpu.SemaphoreType.DMA((NBUF,)),
        ],
    )


def _tc_fused_body(x_ref, g_ref, v_ref, W_ref, b_ref, ls_ref, lb_ref, o_ref):
    x = x_ref[...]
    vb = v_ref[...]
    a = g_ref[0] * vb[0][:, None]
    bb = g_ref[1] * vb[1][:, None]
    c = g_ref[2] * vb[2][:, None]
    d = g_ref[3] * vb[3][:, None]
    patch = jnp.concatenate(
        [x, jnp.abs(a - c), a + c, jnp.abs(bb - d), bb + d], axis=-1)
    y = jnp.dot(patch, W_ref[...], preferred_element_type=jnp.float32)
    y = y + b_ref[...]
    mu = jnp.mean(y, axis=-1, keepdims=True)
    yc = y - mu
    var = jnp.mean(yc * yc, axis=-1, keepdims=True)
    y = yc * lax.rsqrt(var + 1e-5) * ls_ref[...] + lb_ref[...]
    y = y + x
    o_ref[...] = 0.5 * y * (1.0 + lax.erf(y * 0.7071067811865476))


def kernel(x, col1, col2, col3, col4, val1, val2, val3, val4, W, b, ln_scale,
           ln_bias):
    cols = jnp.stack([col1, col2, col3, col4]).astype(jnp.int32)
    cols = jnp.pad(cols, ((0, 0), (0, NP - N)))
    g = _sc_gather()(cols, x)

    vals = jnp.stack([val1, val2, val3, val4])
    vals = jnp.pad(vals, ((0, 0), (0, NP - N)))

    grid = (pl.cdiv(N, BLK),)
    out = pl.pallas_call(
        _tc_fused_body,
        grid=grid,
        in_specs=[
            pl.BlockSpec((BLK, C), lambda i: (i, 0)),
            pl.BlockSpec((4, BLK, C), lambda i: (0, i, 0)),
            pl.BlockSpec((4, BLK), lambda i: (0, i)),
            pl.BlockSpec((5 * C, C), lambda i: (0, 0)),
            pl.BlockSpec((1, C), lambda i: (0, 0)),
            pl.BlockSpec((1, C), lambda i: (0, 0)),
            pl.BlockSpec((1, C), lambda i: (0, 0)),
        ],
        out_specs=pl.BlockSpec((BLK, C), lambda i: (i, 0)),
        out_shape=jax.ShapeDtypeStruct((N, C), jnp.float32),
    )(x, g, vals, W, b[None, :], ln_scale[None, :], ln_bias[None, :])
    return out
